# baseline (speedup 1.0000x reference)
import functools

import jax
import jax.numpy as jnp
from jax import lax
from jax.experimental import pallas as pl
from jax.experimental.pallas import tpu as pltpu
from jax.experimental.pallas import tpu_sc as plsc

_NOISE = 0.5

_B, _H, _W, _C = 8, 224, 224, 192
_ROWS = _B * _H * _W
_NW = 32
_PER_W = _ROWS // _NW
_CHUNK = 128
_NCH = _PER_W // _CHUNK
_NBUF = 2


def _flat_indices():
    ii, jj = jnp.meshgrid(jnp.arange(_H, dtype=jnp.float32),
                          jnp.arange(_W, dtype=jnp.float32), indexing='ij')
    k1, k2 = jax.random.split(jax.random.key(123))
    n_i = jax.random.normal(k1, (_B, _H, _W, 1), dtype=jnp.float32)
    n_j = jax.random.normal(k2, (_B, _H, _W, 1), dtype=jnp.float32)
    a_i = ii[None, :, :, None] + n_i * _NOISE
    a_j = jj[None, :, :, None] + n_j * _NOISE
    a = jnp.concatenate([a_i, a_j], axis=3)
    a = jnp.floor(a + 0.4999).astype(jnp.int32)
    a = jnp.where(a < 0, 0, a)
    s = jnp.array([_H - 1, _W - 1], dtype=jnp.int32)
    a = jnp.where(a > s, s, a)
    flat = a[..., 0] * _W + a[..., 1]
    base = jnp.arange(_B, dtype=jnp.int32) * (_H * _W)
    return (flat + base[:, None, None]).reshape(_ROWS)


def _sc_gather(table, idx):
    mesh = plsc.VectorSubcoreMesh(core_axis_name="c", subcore_axis_name="s")

    @functools.partial(
        pl.kernel,
        out_type=jax.ShapeDtypeStruct((_ROWS, _C), jnp.float32),
        mesh=mesh,
        scratch_types=[
            pltpu.VMEM((_NCH, _CHUNK), jnp.int32),
            pltpu.VMEM((_NBUF, _CHUNK, _C), jnp.float32),
            pltpu.SemaphoreType.DMA,
            pltpu.SemaphoreType.DMA,
        ],
        compiler_params=pltpu.CompilerParams(use_tc_tiling_on_sc=False),
    )
    def k(table_hbm, idx_hbm, out_hbm, idx_v, rows_v, gsem, ssem):
        wid = lax.axis_index("s") * 2 + lax.axis_index("c")
        base = wid * _PER_W
        pltpu.sync_copy(idx_hbm.at[wid], idx_v)

        for b in range(_NBUF):
            pltpu.async_copy(table_hbm.at[idx_v.at[b]], rows_v.at[b], gsem)

        def body(j, _):
            b = lax.rem(j, _NBUF)
            pltpu.make_async_copy(table_hbm.at[idx_v.at[b]], rows_v.at[b],
                                  gsem).wait()
            pltpu.async_copy(
                rows_v.at[b], out_hbm.at[pl.ds(base + j * _CHUNK, _CHUNK)], ssem)

            @pl.when(j + _NBUF < _NCH)
            def _():
                pltpu.make_async_copy(
                    rows_v.at[b],
                    out_hbm.at[pl.ds(base + j * _CHUNK, _CHUNK)], ssem).wait()
                pltpu.async_copy(table_hbm.at[idx_v.at[j + _NBUF]],
                                 rows_v.at[b], gsem)
            return 0

        lax.fori_loop(0, _NCH, body, 0)
        for b in range(_NBUF):
            pltpu.make_async_copy(
                rows_v.at[b],
                out_hbm.at[pl.ds(base + (_NCH - _NBUF + b) * _CHUNK, _CHUNK)],
                ssem).wait()

    return k(table, idx)


def kernel(image):
    idx = _flat_indices().reshape(_NW, _NCH, _CHUNK)
    table = image.reshape(_ROWS, _C)
    out = _sc_gather(table, idx)
    return out.reshape(_B, _H, _W, _C)
